# baseline (speedup 1.0000x reference)
import functools

import jax
import jax.numpy as jnp
from jax import lax
from jax.experimental import pallas as pl
from jax.experimental.pallas import tpu as pltpu
from jax.experimental.pallas import tpu_sc as plsc

N_NODES = 10000
N_PAD = 10240
N_EDGES = 320000
N_QUERY = 100000
D = 128

NC = 2
NS = 16
NW = NC * NS

E_PAD = 327680
E_ROWS = E_PAD // 128
ER_PER_TILE = E_ROWS // NW
EBLK = 8
ROWS_PER_TILE = N_PAD // NS

DEG_PER_TILE = N_PAD // NS

Q_PAD = 102400
Q_ROWS = Q_PAD // 128
QR_PER_TILE = Q_ROWS // NW


def _wid():
    return lax.axis_index("s") * NC + lax.axis_index("c")


def _sc_degree_body(dst2d_hbm, ones_hbm, out_hbm, idxb, onesv, zv, acc_sh, sdeg):
    c = lax.axis_index("c")
    s = lax.axis_index("s")
    wid = _wid()
    for k in range(0, DEG_PER_TILE, 16):
        zv[pl.ds(k, 16)] = jnp.zeros((16,), jnp.float32)
    pltpu.sync_copy(zv, acc_sh.at[pl.ds(s * DEG_PER_TILE, DEG_PER_TILE)])
    pltpu.sync_copy(ones_hbm, onesv)
    plsc.subcore_barrier()

    base = wid * ER_PER_TILE
    pltpu.sync_copy(dst2d_hbm.at[pl.ds(base, ER_PER_TILE)], idxb)
    descs = [
        pltpu.async_copy(onesv, acc_sh.at[idxb.at[j]], sdeg, add=True)
        for j in range(ER_PER_TILE)
    ]
    for d in descs:
        d.wait()
    plsc.subcore_barrier()
    pltpu.sync_copy(
        acc_sh.at[pl.ds(s * DEG_PER_TILE, DEG_PER_TILE)],
        out_hbm.at[c, pl.ds(s * DEG_PER_TILE, DEG_PER_TILE)],
    )


def _sc_segsum_body(y_hbm, src2d_hbm, dst2d_hbm, z_hbm, out_hbm,
                    idxs, idxd, rows0, rows1, acc_sh, sg0, sg1, ss0, ss1):
    c = lax.axis_index("c")
    s = lax.axis_index("s")
    wid = _wid()
    row0 = s * ROWS_PER_TILE
    pltpu.sync_copy(z_hbm.at[pl.ds(row0, ROWS_PER_TILE)],
                    acc_sh.at[pl.ds(row0, ROWS_PER_TILE)])
    plsc.subcore_barrier()

    base = wid * ER_PER_TILE
    rows = (rows0, rows1)
    sg = (sg0, sg1)

    def body(i, carry):
        rb = base + i * EBLK
        pltpu.sync_copy(src2d_hbm.at[pl.ds(rb, EBLK)], idxs)
        pltpu.sync_copy(dst2d_hbm.at[pl.ds(rb, EBLK)], idxd)
        gd = [None] * EBLK
        for jj in range(EBLK):
            sl = jj % 2
            gd[jj] = pltpu.async_copy(y_hbm.at[idxs.at[jj]], rows[sl], sg[sl])
            if jj >= 1:
                gd[jj - 1].wait()
                pltpu.sync_copy(rows[(jj - 1) % 2],
                                acc_sh.at[idxd.at[jj - 1]], add=True)
        gd[EBLK - 1].wait()
        pltpu.sync_copy(rows[(EBLK - 1) % 2], acc_sh.at[idxd.at[EBLK - 1]],
                        add=True)
        return carry

    lax.fori_loop(0, ER_PER_TILE // EBLK, body, 0)
    plsc.subcore_barrier()
    pltpu.sync_copy(acc_sh.at[pl.ds(row0, ROWS_PER_TILE)],
                    out_hbm.at[c, pl.ds(row0, ROWS_PER_TILE)])


def _sc_pairgather_body(u_hbm, v_hbm, qs_hbm, qd_hbm, gu_hbm, gv_hbm,
                        idxs, idxd, ru0, ru1, rv0, rv1,
                        sgu0, sgu1, sgv0, sgv1, swu0, swu1, swv0, swv1):
    wid = _wid()
    base = wid * QR_PER_TILE
    pltpu.sync_copy(qs_hbm.at[pl.ds(base * 128, QR_PER_TILE * 128)], idxs)
    pltpu.sync_copy(qd_hbm.at[pl.ds(base * 128, QR_PER_TILE * 128)], idxd)
    ru = (ru0, ru1)
    rv = (rv0, rv1)
    sgu = (sgu0, sgu1)
    sgv = (sgv0, sgv1)
    swu = (swu0, swu1)
    swv = (swv0, swv1)
    def do_row(j, sl):
        g_u = pltpu.async_copy(u_hbm.at[idxs.at[pl.ds(j * 128, 128)]],
                               ru[sl], sgu[sl])
        g_v = pltpu.async_copy(v_hbm.at[idxd.at[pl.ds(j * 128, 128)]],
                               rv[sl], sgv[sl])
        return g_u, g_v

    def write_row(j, sl):
        w_u = pltpu.async_copy(ru[sl], gu_hbm.at[pl.ds((base + j) * 128, 128)],
                               swu[sl])
        w_v = pltpu.async_copy(rv[sl], gv_hbm.at[pl.ds((base + j) * 128, 128)],
                               swv[sl])
        return w_u, w_v

    def body(i, carry):
        j0 = 2 * i
        j1 = 2 * i + 1
        gu0, gv0 = do_row(j0, 0)
        gu1, gv1 = do_row(j1, 1)
        gu0.wait()
        gv0.wait()
        wu0, wv0 = write_row(j0, 0)
        gu1.wait()
        gv1.wait()
        wu1, wv1 = write_row(j1, 1)
        wu0.wait()
        wv0.wait()
        wu1.wait()
        wv1.wait()
        return carry

    lax.fori_loop(0, QR_PER_TILE // 2, body, 0)
    jt = QR_PER_TILE - 1
    gu_t, gv_t = do_row(jt, 0)
    gu_t.wait()
    gv_t.wait()
    wu_t, wv_t = write_row(jt, 0)
    wu_t.wait()
    wv_t.wait()


@functools.cache
def _sc_kernels():
    mesh = plsc.VectorSubcoreMesh(core_axis_name="c", subcore_axis_name="s",
                                  num_cores=NC, num_subcores=NS)
    sc_degree = pl.kernel(
        _sc_degree_body,
        out_type=jax.ShapeDtypeStruct((NC, N_PAD), jnp.float32),
        mesh=mesh,
        scratch_types=[
            pltpu.VMEM((ER_PER_TILE, 128), jnp.int32),
            pltpu.VMEM((128,), jnp.float32),
            pltpu.VMEM((DEG_PER_TILE,), jnp.float32),
            pltpu.VMEM_SHARED((N_PAD,), jnp.float32),
            pltpu.SemaphoreType.DMA,
        ],
    )
    sc_segsum = pl.kernel(
        _sc_segsum_body,
        out_type=jax.ShapeDtypeStruct((NC, N_PAD, D), jnp.float32),
        mesh=mesh,
        scratch_types=[
            pltpu.VMEM((EBLK, 128), jnp.int32),
            pltpu.VMEM((EBLK, 128), jnp.int32),
            pltpu.VMEM((128, D), jnp.float32),
            pltpu.VMEM((128, D), jnp.float32),
            pltpu.VMEM_SHARED((N_PAD, D), jnp.float32),
            pltpu.SemaphoreType.DMA,
            pltpu.SemaphoreType.DMA,
            pltpu.SemaphoreType.DMA,
            pltpu.SemaphoreType.DMA,
        ],
    )
    sc_pairgather = pl.kernel(
        _sc_pairgather_body,
        out_type=[
            jax.ShapeDtypeStruct((Q_PAD, D), jnp.float32),
            jax.ShapeDtypeStruct((Q_PAD, D), jnp.float32),
        ],
        mesh=mesh,
        scratch_types=(
            [
                pltpu.VMEM((QR_PER_TILE * 128,), jnp.int32),
                pltpu.VMEM((QR_PER_TILE * 128,), jnp.int32),
                pltpu.VMEM((128, D), jnp.float32),
                pltpu.VMEM((128, D), jnp.float32),
                pltpu.VMEM((128, D), jnp.float32),
                pltpu.VMEM((128, D), jnp.float32),
            ]
            + [pltpu.SemaphoreType.DMA] * 8
        ),
    )
    return sc_degree, sc_segsum, sc_pairgather


_BN = 1024
_BQ = 2048


def _tc_prescale_body(x_ref, w1_ref, degp_ref, y1_ref, dinv_ref):
    deg = degp_ref[0, :] + degp_ref[1, :] + 1.0
    dinv = lax.rsqrt(deg)
    xw = jnp.dot(x_ref[...], w1_ref[...], preferred_element_type=jnp.float32)
    y1_ref[...] = xw * dinv[:, None]
    dinv_ref[...] = dinv


def _tc_mid_body(y1_ref, p0_ref, p1_ref, dinv_ref, b1_ref, w2_ref, y2_ref):
    dinv = dinv_ref[...]
    h = (p0_ref[...] + p1_ref[...] + y1_ref[...]) * dinv[:, None] + b1_ref[...][None, :]
    h = jnp.maximum(h, 0.0)
    y2_ref[...] = jnp.dot(h, w2_ref[...], preferred_element_type=jnp.float32) * dinv[:, None]


def _tc_final_body(y2_ref, q0_ref, q1_ref, dinv_ref, b2_ref, wfc_ref, bfc_ref,
                   wd1a_ref, wd1b_ref, bd1_ref, u_ref, v_ref):
    dinv = dinv_ref[...]
    h = (q0_ref[...] + q1_ref[...] + y2_ref[...]) * dinv[:, None] + b2_ref[...][None, :]
    h = jnp.maximum(h, 0.0)
    z = jnp.dot(h, wfc_ref[...], preferred_element_type=jnp.float32) + bfc_ref[...][None, :]
    u_ref[...] = jnp.dot(z, wd1a_ref[...], preferred_element_type=jnp.float32) + bd1_ref[...][None, :]
    v_ref[...] = jnp.dot(z, wd1b_ref[...], preferred_element_type=jnp.float32)


def _tc_dec_body(gu_ref, gv_ref, wd2_ref, bd2_ref, wd3_ref, bd3_ref, o_ref):
    t = jnp.maximum(gu_ref[...] + gv_ref[...], 0.0)
    t2 = jnp.dot(t, wd2_ref[...], preferred_element_type=jnp.float32) + bd2_ref[...][None, :]
    t2 = jnp.maximum(t2, 0.0)
    sc = jnp.sum(t2 * wd3_ref[...][None, :], axis=1) + bd3_ref[...]
    o_ref[...] = jax.nn.sigmoid(sc)


def _full(shape):
    return pl.BlockSpec(shape, lambda i: tuple(0 for _ in shape))


def kernel(x, edge_index, query_edges, W1, b1, W2, b2, Wfc, bfc,
           Wd1, bd1, Wd2, bd2, Wd3, bd3):
    src = edge_index[0].astype(jnp.int32)
    dst = edge_index[1].astype(jnp.int32)
    qs = query_edges[0].astype(jnp.int32)
    qd = query_edges[1].astype(jnp.int32)

    epad = E_PAD - N_EDGES
    pad_node = jnp.full((epad,), N_PAD - 1, jnp.int32)
    src2d = jnp.concatenate([src, pad_node]).reshape(E_ROWS, 128)
    dst2d = jnp.concatenate([dst, pad_node]).reshape(E_ROWS, 128)
    qpad = Q_PAD - N_QUERY
    qs1d = jnp.concatenate([qs, jnp.zeros((qpad,), jnp.int32)])
    qd1d = jnp.concatenate([qd, jnp.zeros((qpad,), jnp.int32)])
    ones_row = jnp.ones((128,), jnp.float32)
    xp = jnp.pad(x, ((0, N_PAD - N_NODES), (0, 0)))
    zeros2d = jnp.zeros((N_PAD, D), jnp.float32)

    _sc_degree, _sc_segsum, _sc_pairgather = _sc_kernels()

    degp = _sc_degree(dst2d, ones_row)

    grid_n = N_PAD // _BN
    y1, dinv = pl.pallas_call(
        _tc_prescale_body,
        grid=(grid_n,),
        in_specs=[
            pl.BlockSpec((_BN, D), lambda i: (i, 0)),
            _full((D, D)),
            pl.BlockSpec((NC, _BN), lambda i: (0, i)),
        ],
        out_specs=[
            pl.BlockSpec((_BN, D), lambda i: (i, 0)),
            pl.BlockSpec((_BN,), lambda i: (i,)),
        ],
        out_shape=[
            jax.ShapeDtypeStruct((N_PAD, D), jnp.float32),
            jax.ShapeDtypeStruct((N_PAD,), jnp.float32),
        ],
    )(xp, W1, degp)

    p = _sc_segsum(y1, src2d, dst2d, zeros2d)

    y2 = pl.pallas_call(
        _tc_mid_body,
        grid=(grid_n,),
        in_specs=[
            pl.BlockSpec((_BN, D), lambda i: (i, 0)),
            pl.BlockSpec((_BN, D), lambda i: (i, 0)),
            pl.BlockSpec((_BN, D), lambda i: (i, 0)),
            pl.BlockSpec((_BN,), lambda i: (i,)),
            _full((D,)),
            _full((D, D)),
        ],
        out_specs=pl.BlockSpec((_BN, D), lambda i: (i, 0)),
        out_shape=jax.ShapeDtypeStruct((N_PAD, D), jnp.float32),
    )(y1, p[0], p[1], dinv, b1, W2)

    q = _sc_segsum(y2, src2d, dst2d, zeros2d)

    Wd1a = Wd1[:D]
    Wd1b = Wd1[D:]
    u, v = pl.pallas_call(
        _tc_final_body,
        grid=(grid_n,),
        in_specs=[
            pl.BlockSpec((_BN, D), lambda i: (i, 0)),
            pl.BlockSpec((_BN, D), lambda i: (i, 0)),
            pl.BlockSpec((_BN, D), lambda i: (i, 0)),
            pl.BlockSpec((_BN,), lambda i: (i,)),
            _full((D,)),
            _full((D, D)),
            _full((D,)),
            _full((D, D)),
            _full((D, D)),
            _full((D,)),
        ],
        out_specs=[
            pl.BlockSpec((_BN, D), lambda i: (i, 0)),
            pl.BlockSpec((_BN, D), lambda i: (i, 0)),
        ],
        out_shape=[
            jax.ShapeDtypeStruct((N_PAD, D), jnp.float32),
            jax.ShapeDtypeStruct((N_PAD, D), jnp.float32),
        ],
    )(y2, q[0], q[1], dinv, b2, Wfc, bfc, Wd1a, Wd1b, bd1)

    gu, gv = _sc_pairgather(u, v, qs1d, qd1d)

    grid_q = Q_PAD // _BQ
    out = pl.pallas_call(
        _tc_dec_body,
        grid=(grid_q,),
        in_specs=[
            pl.BlockSpec((_BQ, D), lambda i: (i, 0)),
            pl.BlockSpec((_BQ, D), lambda i: (i, 0)),
            _full((D, D // 2)),
            _full((D // 2,)),
            _full((D // 2,)),
            _full((1,)),
        ],
        out_specs=pl.BlockSpec((_BQ,), lambda i: (i,)),
        out_shape=jax.ShapeDtypeStruct((Q_PAD,), jnp.float32),
    )(gu, gv, Wd2, bd2, Wd3[:, 0], bd3)

    return out[:N_QUERY]


# speedup vs baseline: 1.3401x; 1.3401x over previous
import functools

import jax
import jax.numpy as jnp
from jax import lax
from jax.experimental import pallas as pl
from jax.experimental.pallas import tpu as pltpu
from jax.experimental.pallas import tpu_sc as plsc

N_NODES = 10000
N_PAD = 10240
N_EDGES = 320000
N_QUERY = 100000
D = 128

NC = 2
NS = 16
NW = NC * NS

E_PAD = 327680
E_ROWS = E_PAD // 128
ER_PER_TILE = E_ROWS // NW
EBLK = 8
ROWS_PER_TILE = N_PAD // NS

DEG_PER_TILE = N_PAD // NS

Q_PAD = 102400
Q_ROWS = Q_PAD // 128
QR_PER_TILE = Q_ROWS // NW


def _wid():
    return lax.axis_index("s") * NC + lax.axis_index("c")


def _sc_degree_body(dst2d_hbm, ones_hbm, out_hbm, idxb, onesv, zv, acc_sh, sdeg):
    c = lax.axis_index("c")
    s = lax.axis_index("s")
    wid = _wid()
    for k in range(0, DEG_PER_TILE, 16):
        zv[pl.ds(k, 16)] = jnp.zeros((16,), jnp.float32)
    pltpu.sync_copy(zv, acc_sh.at[pl.ds(s * DEG_PER_TILE, DEG_PER_TILE)])
    pltpu.sync_copy(ones_hbm, onesv)
    plsc.subcore_barrier()

    base = wid * ER_PER_TILE
    pltpu.sync_copy(dst2d_hbm.at[pl.ds(base, ER_PER_TILE)], idxb)
    descs = [
        pltpu.async_copy(onesv, acc_sh.at[idxb.at[j]], sdeg, add=True)
        for j in range(ER_PER_TILE)
    ]
    for d in descs:
        d.wait()
    plsc.subcore_barrier()
    pltpu.sync_copy(
        acc_sh.at[pl.ds(s * DEG_PER_TILE, DEG_PER_TILE)],
        out_hbm.at[c, pl.ds(s * DEG_PER_TILE, DEG_PER_TILE)],
    )


def _sc_segsum_body(y_hbm, src2d_hbm, dst2d_hbm, z_hbm, out_hbm,
                    idxs, idxd, rows0, rows1, acc_sh, sg0, sg1, ss0, ss1):
    c = lax.axis_index("c")
    s = lax.axis_index("s")
    wid = _wid()
    row0 = s * ROWS_PER_TILE
    pltpu.sync_copy(z_hbm.at[pl.ds(row0, ROWS_PER_TILE)],
                    acc_sh.at[pl.ds(row0, ROWS_PER_TILE)])
    plsc.subcore_barrier()

    base = wid * ER_PER_TILE
    rows = (rows0, rows1)
    sg = (sg0, sg1)

    def body(i, carry):
        rb = base + i * EBLK
        pltpu.sync_copy(src2d_hbm.at[pl.ds(rb, EBLK)], idxs)
        pltpu.sync_copy(dst2d_hbm.at[pl.ds(rb, EBLK)], idxd)
        gd = [None] * EBLK
        for jj in range(EBLK):
            sl = jj % 2
            gd[jj] = pltpu.async_copy(y_hbm.at[idxs.at[jj]], rows[sl], sg[sl])
            if jj >= 1:
                gd[jj - 1].wait()
                pltpu.sync_copy(rows[(jj - 1) % 2],
                                acc_sh.at[idxd.at[jj - 1]], add=True)
        gd[EBLK - 1].wait()
        pltpu.sync_copy(rows[(EBLK - 1) % 2], acc_sh.at[idxd.at[EBLK - 1]],
                        add=True)
        return carry

    lax.fori_loop(0, ER_PER_TILE // EBLK, body, 0)
    plsc.subcore_barrier()
    pltpu.sync_copy(acc_sh.at[pl.ds(row0, ROWS_PER_TILE)],
                    out_hbm.at[c, pl.ds(row0, ROWS_PER_TILE)])


def _sc_pairgather_body(u_hbm, v_hbm, qs_hbm, qd_hbm, gu_hbm, gv_hbm,
                        idxs, idxd, ru0, ru1, rv0, rv1,
                        sgu0, sgu1, sgv0, sgv1, swu0, swu1, swv0, swv1):
    wid = _wid()
    base = wid * QR_PER_TILE
    pltpu.sync_copy(qs_hbm.at[pl.ds(base * 128, QR_PER_TILE * 128)], idxs)
    pltpu.sync_copy(qd_hbm.at[pl.ds(base * 128, QR_PER_TILE * 128)], idxd)
    ru = (ru0, ru1)
    rv = (rv0, rv1)
    sgu = (sgu0, sgu1)
    sgv = (sgv0, sgv1)
    swu = (swu0, swu1)
    swv = (swv0, swv1)
    def do_row(j, sl):
        g_u = pltpu.async_copy(u_hbm.at[idxs.at[pl.ds(j * 128, 128)]],
                               ru[sl], sgu[sl])
        g_v = pltpu.async_copy(v_hbm.at[idxd.at[pl.ds(j * 128, 128)]],
                               rv[sl], sgv[sl])
        return g_u, g_v

    def write_row(j, sl):
        w_u = pltpu.async_copy(ru[sl], gu_hbm.at[pl.ds((base + j) * 128, 128)],
                               swu[sl])
        w_v = pltpu.async_copy(rv[sl], gv_hbm.at[pl.ds((base + j) * 128, 128)],
                               swv[sl])
        return w_u, w_v

    def body(i, carry):
        j0 = 2 * i
        j1 = 2 * i + 1
        gu0, gv0 = do_row(j0, 0)
        gu1, gv1 = do_row(j1, 1)
        gu0.wait()
        gv0.wait()
        wu0, wv0 = write_row(j0, 0)
        gu1.wait()
        gv1.wait()
        wu1, wv1 = write_row(j1, 1)
        wu0.wait()
        wv0.wait()
        wu1.wait()
        wv1.wait()
        return carry

    lax.fori_loop(0, QR_PER_TILE // 2, body, 0)
    jt = QR_PER_TILE - 1
    gu_t, gv_t = do_row(jt, 0)
    gu_t.wait()
    gv_t.wait()
    wu_t, wv_t = write_row(jt, 0)
    wu_t.wait()
    wv_t.wait()


@functools.cache
def _sc_kernels():
    mesh = plsc.VectorSubcoreMesh(core_axis_name="c", subcore_axis_name="s",
                                  num_cores=NC, num_subcores=NS)
    sc_degree = pl.kernel(
        _sc_degree_body,
        out_type=jax.ShapeDtypeStruct((NC, N_PAD), jnp.float32),
        mesh=mesh,
        scratch_types=[
            pltpu.VMEM((ER_PER_TILE, 128), jnp.int32),
            pltpu.VMEM((128,), jnp.float32),
            pltpu.VMEM((DEG_PER_TILE,), jnp.float32),
            pltpu.VMEM_SHARED((N_PAD,), jnp.float32),
            pltpu.SemaphoreType.DMA,
        ],
    )
    sc_segsum = pl.kernel(
        _sc_segsum_body,
        out_type=jax.ShapeDtypeStruct((NC, N_PAD, D), jnp.float32),
        mesh=mesh,
        scratch_types=[
            pltpu.VMEM((EBLK, 128), jnp.int32),
            pltpu.VMEM((EBLK, 128), jnp.int32),
            pltpu.VMEM((128, D), jnp.float32),
            pltpu.VMEM((128, D), jnp.float32),
            pltpu.VMEM_SHARED((N_PAD, D), jnp.float32),
            pltpu.SemaphoreType.DMA,
            pltpu.SemaphoreType.DMA,
            pltpu.SemaphoreType.DMA,
            pltpu.SemaphoreType.DMA,
        ],
    )
    sc_pairgather = pl.kernel(
        _sc_pairgather_body,
        out_type=[
            jax.ShapeDtypeStruct((Q_PAD, D), jnp.float32),
            jax.ShapeDtypeStruct((Q_PAD, D), jnp.float32),
        ],
        mesh=mesh,
        scratch_types=(
            [
                pltpu.VMEM((QR_PER_TILE * 128,), jnp.int32),
                pltpu.VMEM((QR_PER_TILE * 128,), jnp.int32),
                pltpu.VMEM((128, D), jnp.float32),
                pltpu.VMEM((128, D), jnp.float32),
                pltpu.VMEM((128, D), jnp.float32),
                pltpu.VMEM((128, D), jnp.float32),
            ]
            + [pltpu.SemaphoreType.DMA] * 8
        ),
    )
    return sc_degree, sc_segsum, sc_pairgather


_BN = 1024
_BQ = 2048


def _tc_prescale_body(x_ref, w1_ref, degp_ref, y1_ref, dinv_ref):
    deg = degp_ref[0, :] + degp_ref[1, :] + 1.0
    dinv = lax.rsqrt(deg)
    xw = jnp.dot(x_ref[...], w1_ref[...], preferred_element_type=jnp.float32)
    y1_ref[...] = xw * dinv[:, None]
    dinv_ref[...] = dinv


def _tc_mid_body(y1_ref, p0_ref, p1_ref, dinv_ref, b1_ref, w2_ref, y2_ref):
    dinv = dinv_ref[...]
    h = (p0_ref[...] + p1_ref[...] + y1_ref[...]) * dinv[:, None] + b1_ref[...][None, :]
    h = jnp.maximum(h, 0.0)
    y2_ref[...] = jnp.dot(h, w2_ref[...], preferred_element_type=jnp.float32) * dinv[:, None]


def _tc_final_body(y2_ref, q0_ref, q1_ref, dinv_ref, b2_ref, wfc_ref, bfc_ref,
                   wd1a_ref, wd1b_ref, bd1_ref, u_ref, v_ref):
    dinv = dinv_ref[...]
    h = (q0_ref[...] + q1_ref[...] + y2_ref[...]) * dinv[:, None] + b2_ref[...][None, :]
    h = jnp.maximum(h, 0.0)
    z = jnp.dot(h, wfc_ref[...], preferred_element_type=jnp.float32) + bfc_ref[...][None, :]
    u_ref[...] = jnp.dot(z, wd1a_ref[...], preferred_element_type=jnp.float32) + bd1_ref[...][None, :]
    v_ref[...] = jnp.dot(z, wd1b_ref[...], preferred_element_type=jnp.float32)


def _tc_dec_body(gu_ref, gv_ref, wd2_ref, bd2_ref, wd3_ref, bd3_ref, o_ref):
    t = jnp.maximum(gu_ref[...] + gv_ref[...], 0.0)
    t2 = jnp.dot(t, wd2_ref[...], preferred_element_type=jnp.float32) + bd2_ref[...][None, :]
    t2 = jnp.maximum(t2, 0.0)
    sc = jnp.sum(t2 * wd3_ref[...][None, :], axis=1) + bd3_ref[...]
    o_ref[...] = jax.nn.sigmoid(sc)


def _full(shape):
    return pl.BlockSpec(shape, lambda i: tuple(0 for _ in shape))


def kernel(x, edge_index, query_edges, W1, b1, W2, b2, Wfc, bfc,
           Wd1, bd1, Wd2, bd2, Wd3, bd3):
    src = edge_index[0].astype(jnp.int32)
    dst = edge_index[1].astype(jnp.int32)
    qs = query_edges[0].astype(jnp.int32)
    qd = query_edges[1].astype(jnp.int32)

    epad = E_PAD - N_EDGES
    pad_src = jnp.full((epad,), N_PAD - 1, jnp.int32)
    pad_dst = (jnp.arange(epad, dtype=jnp.int32) % (N_PAD - N_NODES)) + N_NODES
    src2d = jnp.concatenate([src, pad_src]).reshape(E_ROWS, 128)
    dst2d = jnp.concatenate([dst, pad_dst]).reshape(E_ROWS, 128)
    qpad = Q_PAD - N_QUERY
    pad_q = jnp.arange(qpad, dtype=jnp.int32) % N_PAD
    qs1d = jnp.concatenate([qs, pad_q])
    qd1d = jnp.concatenate([qd, pad_q])
    ones_row = jnp.ones((128,), jnp.float32)
    xp = jnp.pad(x, ((0, N_PAD - N_NODES), (0, 0)))
    zeros2d = jnp.zeros((N_PAD, D), jnp.float32)

    _sc_degree, _sc_segsum, _sc_pairgather = _sc_kernels()

    degp = _sc_degree(dst2d, ones_row)

    grid_n = N_PAD // _BN
    y1, dinv = pl.pallas_call(
        _tc_prescale_body,
        grid=(grid_n,),
        in_specs=[
            pl.BlockSpec((_BN, D), lambda i: (i, 0)),
            _full((D, D)),
            pl.BlockSpec((NC, _BN), lambda i: (0, i)),
        ],
        out_specs=[
            pl.BlockSpec((_BN, D), lambda i: (i, 0)),
            pl.BlockSpec((_BN,), lambda i: (i,)),
        ],
        out_shape=[
            jax.ShapeDtypeStruct((N_PAD, D), jnp.float32),
            jax.ShapeDtypeStruct((N_PAD,), jnp.float32),
        ],
    )(xp, W1, degp)

    p = _sc_segsum(y1, src2d, dst2d, zeros2d)

    y2 = pl.pallas_call(
        _tc_mid_body,
        grid=(grid_n,),
        in_specs=[
            pl.BlockSpec((_BN, D), lambda i: (i, 0)),
            pl.BlockSpec((_BN, D), lambda i: (i, 0)),
            pl.BlockSpec((_BN, D), lambda i: (i, 0)),
            pl.BlockSpec((_BN,), lambda i: (i,)),
            _full((D,)),
            _full((D, D)),
        ],
        out_specs=pl.BlockSpec((_BN, D), lambda i: (i, 0)),
        out_shape=jax.ShapeDtypeStruct((N_PAD, D), jnp.float32),
    )(y1, p[0], p[1], dinv, b1, W2)

    q = _sc_segsum(y2, src2d, dst2d, zeros2d)

    Wd1a = Wd1[:D]
    Wd1b = Wd1[D:]
    u, v = pl.pallas_call(
        _tc_final_body,
        grid=(grid_n,),
        in_specs=[
            pl.BlockSpec((_BN, D), lambda i: (i, 0)),
            pl.BlockSpec((_BN, D), lambda i: (i, 0)),
            pl.BlockSpec((_BN, D), lambda i: (i, 0)),
            pl.BlockSpec((_BN,), lambda i: (i,)),
            _full((D,)),
            _full((D, D)),
            _full((D,)),
            _full((D, D)),
            _full((D, D)),
            _full((D,)),
        ],
        out_specs=[
            pl.BlockSpec((_BN, D), lambda i: (i, 0)),
            pl.BlockSpec((_BN, D), lambda i: (i, 0)),
        ],
        out_shape=[
            jax.ShapeDtypeStruct((N_PAD, D), jnp.float32),
            jax.ShapeDtypeStruct((N_PAD, D), jnp.float32),
        ],
    )(y2, q[0], q[1], dinv, b2, Wfc, bfc, Wd1a, Wd1b, bd1)

    gu, gv = _sc_pairgather(u, v, qs1d, qd1d)

    grid_q = Q_PAD // _BQ
    out = pl.pallas_call(
        _tc_dec_body,
        grid=(grid_q,),
        in_specs=[
            pl.BlockSpec((_BQ, D), lambda i: (i, 0)),
            pl.BlockSpec((_BQ, D), lambda i: (i, 0)),
            _full((D, D // 2)),
            _full((D // 2,)),
            _full((D // 2,)),
            _full((1,)),
        ],
        out_specs=pl.BlockSpec((_BQ,), lambda i: (i,)),
        out_shape=jax.ShapeDtypeStruct((Q_PAD,), jnp.float32),
    )(gu, gv, Wd2, bd2, Wd3[:, 0], bd3)

    return out[:N_QUERY]


# speedup vs baseline: 2.9219x; 2.1804x over previous
import functools

import jax
import jax.numpy as jnp
from jax import lax
from jax.experimental import pallas as pl
from jax.experimental.pallas import tpu as pltpu
from jax.experimental.pallas import tpu_sc as plsc

N_NODES = 10000
N_PAD = 10240
N_EDGES = 320000
N_QUERY = 100000
D = 128

NC = 2
NS = 16
NW = NC * NS

E_PAD = 327680
E_ROWS = E_PAD // 128
ER_PER_TILE = E_ROWS // NW
EBLK = 8
ROWS_PER_TILE = N_PAD // NS

DEG_PER_TILE = N_PAD // NS

Q_PAD = 102400
Q_ROWS = Q_PAD // 128
QR_PER_TILE = Q_ROWS // NW


def _wid():
    return lax.axis_index("s") * NC + lax.axis_index("c")


def _sc_degree_body(dst2d_hbm, ones_hbm, out_hbm, idxb, onesv, zv, acc_sh, sdeg):
    c = lax.axis_index("c")
    s = lax.axis_index("s")
    wid = _wid()
    for k in range(0, DEG_PER_TILE, 16):
        zv[pl.ds(k, 16)] = jnp.zeros((16,), jnp.float32)
    pltpu.sync_copy(zv, acc_sh.at[pl.ds(s * DEG_PER_TILE, DEG_PER_TILE)])
    pltpu.sync_copy(ones_hbm, onesv)
    plsc.subcore_barrier()

    base = wid * ER_PER_TILE
    pltpu.sync_copy(dst2d_hbm.at[pl.ds(base, ER_PER_TILE)], idxb)
    descs = [
        pltpu.async_copy(onesv, acc_sh.at[idxb.at[j]], sdeg, add=True)
        for j in range(ER_PER_TILE)
    ]
    for d in descs:
        d.wait()
    plsc.subcore_barrier()
    pltpu.sync_copy(
        acc_sh.at[pl.ds(s * DEG_PER_TILE, DEG_PER_TILE)],
        out_hbm.at[c, pl.ds(s * DEG_PER_TILE, DEG_PER_TILE)],
    )


def _sc_segsum_body(y_hbm, src2d_hbm, dst2d_hbm, z_hbm, out_hbm,
                    idxs, idxd, rows0, rows1, acc_sh, sg0, sg1, ss0, ss1):
    c = lax.axis_index("c")
    s = lax.axis_index("s")
    wid = _wid()
    row0 = s * ROWS_PER_TILE
    pltpu.sync_copy(z_hbm.at[pl.ds(row0, ROWS_PER_TILE)],
                    acc_sh.at[pl.ds(row0, ROWS_PER_TILE)])
    plsc.subcore_barrier()

    base = wid * ER_PER_TILE
    rows = (rows0, rows1)
    sg = (sg0, sg1)

    def body(i, carry):
        rb = base + i * EBLK
        pltpu.sync_copy(src2d_hbm.at[pl.ds(rb, EBLK)], idxs)
        pltpu.sync_copy(dst2d_hbm.at[pl.ds(rb, EBLK)], idxd)
        gd = [None] * EBLK
        for jj in range(EBLK):
            sl = jj % 2
            gd[jj] = pltpu.async_copy(y_hbm.at[idxs.at[jj]], rows[sl], sg[sl])
            if jj >= 1:
                gd[jj - 1].wait()
                pltpu.sync_copy(rows[(jj - 1) % 2],
                                acc_sh.at[idxd.at[jj - 1]], add=True)
        gd[EBLK - 1].wait()
        pltpu.sync_copy(rows[(EBLK - 1) % 2], acc_sh.at[idxd.at[EBLK - 1]],
                        add=True)
        return carry

    lax.fori_loop(0, ER_PER_TILE // EBLK, body, 0)
    plsc.subcore_barrier()
    pltpu.sync_copy(acc_sh.at[pl.ds(row0, ROWS_PER_TILE)],
                    out_hbm.at[c, pl.ds(row0, ROWS_PER_TILE)])


def _sc_pairgather_body(u_hbm, v_hbm, qs_hbm, qd_hbm, gu_hbm, gv_hbm,
                        idxs, idxd, ru0, ru1, rv0, rv1,
                        sgu0, sgu1, sgv0, sgv1, swu0, swu1, swv0, swv1):
    wid = _wid()
    base = wid * QR_PER_TILE
    pltpu.sync_copy(qs_hbm.at[pl.ds(base * 128, QR_PER_TILE * 128)], idxs)
    pltpu.sync_copy(qd_hbm.at[pl.ds(base * 128, QR_PER_TILE * 128)], idxd)
    ru = (ru0, ru1)
    rv = (rv0, rv1)
    sgu = (sgu0, sgu1)
    sgv = (sgv0, sgv1)
    swu = (swu0, swu1)
    swv = (swv0, swv1)
    def do_row(j, sl):
        g_u = pltpu.async_copy(u_hbm.at[idxs.at[pl.ds(j * 128, 128)]],
                               ru[sl], sgu[sl])
        g_v = pltpu.async_copy(v_hbm.at[idxd.at[pl.ds(j * 128, 128)]],
                               rv[sl], sgv[sl])
        return g_u, g_v

    def write_row(j, sl):
        w_u = pltpu.async_copy(ru[sl], gu_hbm.at[pl.ds((base + j) * 128, 128)],
                               swu[sl])
        w_v = pltpu.async_copy(rv[sl], gv_hbm.at[pl.ds((base + j) * 128, 128)],
                               swv[sl])
        return w_u, w_v

    def body(i, carry):
        j0 = 2 * i
        j1 = 2 * i + 1
        gu0, gv0 = do_row(j0, 0)
        gu1, gv1 = do_row(j1, 1)
        gu0.wait()
        gv0.wait()
        wu0, wv0 = write_row(j0, 0)
        gu1.wait()
        gv1.wait()
        wu1, wv1 = write_row(j1, 1)
        wu0.wait()
        wv0.wait()
        wu1.wait()
        wv1.wait()
        return carry

    lax.fori_loop(0, QR_PER_TILE // 2, body, 0)
    jt = QR_PER_TILE - 1
    gu_t, gv_t = do_row(jt, 0)
    gu_t.wait()
    gv_t.wait()
    wu_t, wv_t = write_row(jt, 0)
    wu_t.wait()
    wv_t.wait()


@functools.cache
def _sc_kernels():
    mesh = plsc.VectorSubcoreMesh(core_axis_name="c", subcore_axis_name="s",
                                  num_cores=NC, num_subcores=NS)
    sc_degree = pl.kernel(
        _sc_degree_body,
        out_type=jax.ShapeDtypeStruct((NC, N_PAD), jnp.float32),
        mesh=mesh,
        scratch_types=[
            pltpu.VMEM((ER_PER_TILE, 128), jnp.int32),
            pltpu.VMEM((128,), jnp.float32),
            pltpu.VMEM((DEG_PER_TILE,), jnp.float32),
            pltpu.VMEM_SHARED((N_PAD,), jnp.float32),
            pltpu.SemaphoreType.DMA,
        ],
    )
    sc_segsum = pl.kernel(
        _sc_segsum_body,
        out_type=jax.ShapeDtypeStruct((NC, N_PAD, D), jnp.float32),
        mesh=mesh,
        scratch_types=[
            pltpu.VMEM((EBLK, 128), jnp.int32),
            pltpu.VMEM((EBLK, 128), jnp.int32),
            pltpu.VMEM((128, D), jnp.float32),
            pltpu.VMEM((128, D), jnp.float32),
            pltpu.VMEM_SHARED((N_PAD, D), jnp.float32),
            pltpu.SemaphoreType.DMA,
            pltpu.SemaphoreType.DMA,
            pltpu.SemaphoreType.DMA,
            pltpu.SemaphoreType.DMA,
        ],
    )
    sc_pairgather = pl.kernel(
        _sc_pairgather_body,
        out_type=[
            jax.ShapeDtypeStruct((Q_PAD, D), jnp.float32),
            jax.ShapeDtypeStruct((Q_PAD, D), jnp.float32),
        ],
        mesh=mesh,
        scratch_types=(
            [
                pltpu.VMEM((QR_PER_TILE * 128,), jnp.int32),
                pltpu.VMEM((QR_PER_TILE * 128,), jnp.int32),
                pltpu.VMEM((128, D), jnp.float32),
                pltpu.VMEM((128, D), jnp.float32),
                pltpu.VMEM((128, D), jnp.float32),
                pltpu.VMEM((128, D), jnp.float32),
            ]
            + [pltpu.SemaphoreType.DMA] * 8
        ),
    )
    return sc_degree, sc_segsum, sc_pairgather


_BN = 1024
_BQ = 2048


def _tc_prescale_body(x_ref, w1_ref, degp_ref, y1_ref, dinv_ref):
    deg = degp_ref[0, :] + degp_ref[1, :] + 1.0
    dinv = lax.rsqrt(deg)
    xw = jnp.dot(x_ref[...], w1_ref[...], preferred_element_type=jnp.float32)
    y1_ref[...] = xw * dinv[:, None]
    dinv_ref[...] = dinv


def _tc_mid_body(y1_ref, p0_ref, p1_ref, dinv_ref, b1_ref, w2_ref, y2_ref):
    dinv = dinv_ref[...]
    h = (p0_ref[...] + p1_ref[...] + y1_ref[...]) * dinv[:, None] + b1_ref[...][None, :]
    h = jnp.maximum(h, 0.0)
    y2_ref[...] = jnp.dot(h, w2_ref[...], preferred_element_type=jnp.float32) * dinv[:, None]


def _tc_final_body(y2_ref, q0_ref, q1_ref, dinv_ref, b2_ref, wfc_ref, bfc_ref,
                   wd1a_ref, wd1b_ref, bd1_ref, u_ref, v_ref):
    dinv = dinv_ref[...]
    h = (q0_ref[...] + q1_ref[...] + y2_ref[...]) * dinv[:, None] + b2_ref[...][None, :]
    h = jnp.maximum(h, 0.0)
    z = jnp.dot(h, wfc_ref[...], preferred_element_type=jnp.float32) + bfc_ref[...][None, :]
    u_ref[...] = jnp.dot(z, wd1a_ref[...], preferred_element_type=jnp.float32) + bd1_ref[...][None, :]
    v_ref[...] = jnp.dot(z, wd1b_ref[...], preferred_element_type=jnp.float32)


def _tc_dec_body(gu_ref, gv_ref, wd2_ref, bd2_ref, wd3_ref, bd3_ref, o_ref):
    t = jnp.maximum(gu_ref[...] + gv_ref[...], 0.0)
    t2 = jnp.dot(t, wd2_ref[...], preferred_element_type=jnp.float32) + bd2_ref[...][None, :]
    t2 = jnp.maximum(t2, 0.0)
    sc = jnp.sum(t2 * wd3_ref[...][None, :], axis=1) + bd3_ref[...]
    o_ref[...] = jax.nn.sigmoid(sc)


def _full(shape):
    return pl.BlockSpec(shape, lambda i: tuple(0 for _ in shape))


def kernel(x, edge_index, query_edges, W1, b1, W2, b2, Wfc, bfc,
           Wd1, bd1, Wd2, bd2, Wd3, bd3):
    src = edge_index[0].astype(jnp.int32)
    dst = edge_index[1].astype(jnp.int32)
    qs = query_edges[0].astype(jnp.int32)
    qd = query_edges[1].astype(jnp.int32)

    epad = E_PAD - N_EDGES
    pad_src = jnp.arange(epad, dtype=jnp.int32) % N_PAD
    pad_dst = (jnp.arange(epad, dtype=jnp.int32) % (N_PAD - N_NODES)) + N_NODES
    src2d = jnp.concatenate([src, pad_src]).reshape(E_ROWS, 128)
    dst2d = jnp.concatenate([dst, pad_dst]).reshape(E_ROWS, 128)
    qpad = Q_PAD - N_QUERY
    pad_q = jnp.arange(qpad, dtype=jnp.int32) % N_PAD
    qs1d = jnp.concatenate([qs, pad_q])
    qd1d = jnp.concatenate([qd, pad_q])
    ones_row = jnp.ones((128,), jnp.float32)
    xp = jnp.pad(x, ((0, N_PAD - N_NODES), (0, 0)))
    zeros2d = jnp.zeros((N_PAD, D), jnp.float32)

    _sc_degree, _sc_segsum, _sc_pairgather = _sc_kernels()

    degp = _sc_degree(dst2d, ones_row)

    grid_n = N_PAD // _BN
    y1, dinv = pl.pallas_call(
        _tc_prescale_body,
        grid=(grid_n,),
        in_specs=[
            pl.BlockSpec((_BN, D), lambda i: (i, 0)),
            _full((D, D)),
            pl.BlockSpec((NC, _BN), lambda i: (0, i)),
        ],
        out_specs=[
            pl.BlockSpec((_BN, D), lambda i: (i, 0)),
            pl.BlockSpec((_BN,), lambda i: (i,)),
        ],
        out_shape=[
            jax.ShapeDtypeStruct((N_PAD, D), jnp.float32),
            jax.ShapeDtypeStruct((N_PAD,), jnp.float32),
        ],
    )(xp, W1, degp)

    p = _sc_segsum(y1, src2d, dst2d, zeros2d)

    y2 = pl.pallas_call(
        _tc_mid_body,
        grid=(grid_n,),
        in_specs=[
            pl.BlockSpec((_BN, D), lambda i: (i, 0)),
            pl.BlockSpec((_BN, D), lambda i: (i, 0)),
            pl.BlockSpec((_BN, D), lambda i: (i, 0)),
            pl.BlockSpec((_BN,), lambda i: (i,)),
            _full((D,)),
            _full((D, D)),
        ],
        out_specs=pl.BlockSpec((_BN, D), lambda i: (i, 0)),
        out_shape=jax.ShapeDtypeStruct((N_PAD, D), jnp.float32),
    )(y1, p[0], p[1], dinv, b1, W2)

    q = _sc_segsum(y2, src2d, dst2d, zeros2d)

    Wd1a = Wd1[:D]
    Wd1b = Wd1[D:]
    u, v = pl.pallas_call(
        _tc_final_body,
        grid=(grid_n,),
        in_specs=[
            pl.BlockSpec((_BN, D), lambda i: (i, 0)),
            pl.BlockSpec((_BN, D), lambda i: (i, 0)),
            pl.BlockSpec((_BN, D), lambda i: (i, 0)),
            pl.BlockSpec((_BN,), lambda i: (i,)),
            _full((D,)),
            _full((D, D)),
            _full((D,)),
            _full((D, D)),
            _full((D, D)),
            _full((D,)),
        ],
        out_specs=[
            pl.BlockSpec((_BN, D), lambda i: (i, 0)),
            pl.BlockSpec((_BN, D), lambda i: (i, 0)),
        ],
        out_shape=[
            jax.ShapeDtypeStruct((N_PAD, D), jnp.float32),
            jax.ShapeDtypeStruct((N_PAD, D), jnp.float32),
        ],
    )(y2, q[0], q[1], dinv, b2, Wfc, bfc, Wd1a, Wd1b, bd1)

    gu, gv = _sc_pairgather(u, v, qs1d, qd1d)

    grid_q = Q_PAD // _BQ
    out = pl.pallas_call(
        _tc_dec_body,
        grid=(grid_q,),
        in_specs=[
            pl.BlockSpec((_BQ, D), lambda i: (i, 0)),
            pl.BlockSpec((_BQ, D), lambda i: (i, 0)),
            _full((D, D // 2)),
            _full((D // 2,)),
            _full((D // 2,)),
            _full((1,)),
        ],
        out_specs=pl.BlockSpec((_BQ,), lambda i: (i,)),
        out_shape=jax.ShapeDtypeStruct((Q_PAD,), jnp.float32),
    )(gu, gv, Wd2, bd2, Wd3[:, 0], bd3)

    return out[:N_QUERY]


# speedup vs baseline: 2.9258x; 1.0013x over previous
import functools

import jax
import jax.numpy as jnp
from jax import lax
from jax.experimental import pallas as pl
from jax.experimental.pallas import tpu as pltpu
from jax.experimental.pallas import tpu_sc as plsc

N_NODES = 10000
N_PAD = 10240
N_EDGES = 320000
N_QUERY = 100000
D = 128

NC = 2
NS = 16
NW = NC * NS

E_PAD = 327680
E_ROWS = E_PAD // 128
ER_PER_TILE = E_ROWS // NW
EBLK = 8
ROWS_PER_TILE = N_PAD // NS

DEG_PER_TILE = N_PAD // NS

Q_PAD = 102400
Q_ROWS = Q_PAD // 128
QR_PER_TILE = Q_ROWS // NW


def _wid():
    return lax.axis_index("s") * NC + lax.axis_index("c")


def _sc_degree_body(dst2d_hbm, ones_hbm, out_hbm, idxb, onesv, zv, acc_sh, sdeg):
    c = lax.axis_index("c")
    s = lax.axis_index("s")
    wid = _wid()
    for k in range(0, DEG_PER_TILE, 16):
        zv[pl.ds(k, 16)] = jnp.zeros((16,), jnp.float32)
    pltpu.sync_copy(zv, acc_sh.at[pl.ds(s * DEG_PER_TILE, DEG_PER_TILE)])
    pltpu.sync_copy(ones_hbm, onesv)
    plsc.subcore_barrier()

    base = wid * ER_PER_TILE
    pltpu.sync_copy(dst2d_hbm.at[pl.ds(base, ER_PER_TILE)], idxb)
    descs = [
        pltpu.async_copy(onesv, acc_sh.at[idxb.at[j]], sdeg, add=True)
        for j in range(ER_PER_TILE)
    ]
    for d in descs:
        d.wait()
    plsc.subcore_barrier()
    pltpu.sync_copy(
        acc_sh.at[pl.ds(s * DEG_PER_TILE, DEG_PER_TILE)],
        out_hbm.at[c, pl.ds(s * DEG_PER_TILE, DEG_PER_TILE)],
    )


def _sc_segsum_body(y_hbm, src2d_hbm, dst2d_hbm, z_hbm, out_hbm,
                    idxs, idxd, rows0, rows1, acc_sh, sg0, sg1, ss0, ss1):
    c = lax.axis_index("c")
    s = lax.axis_index("s")
    wid = _wid()
    row0 = s * ROWS_PER_TILE
    pltpu.sync_copy(z_hbm.at[pl.ds(row0, ROWS_PER_TILE)],
                    acc_sh.at[pl.ds(row0, ROWS_PER_TILE)])
    plsc.subcore_barrier()

    base = wid * ER_PER_TILE
    rows = (rows0, rows1)
    sg = (sg0, sg1)

    ss = (ss0, ss1)

    def body(i, carry):
        rb = base + i * EBLK
        pltpu.sync_copy(src2d_hbm.at[pl.ds(rb, EBLK)], idxs)
        pltpu.sync_copy(dst2d_hbm.at[pl.ds(rb, EBLK)], idxd)
        gd = [None] * EBLK
        sd = [None] * EBLK
        for jj in range(EBLK):
            sl = jj % 2
            if jj >= 2:
                sd[jj - 2].wait()
            gd[jj] = pltpu.async_copy(y_hbm.at[idxs.at[jj]], rows[sl], sg[sl])
            if jj >= 1:
                gd[jj - 1].wait()
                sd[jj - 1] = pltpu.async_copy(rows[(jj - 1) % 2],
                                              acc_sh.at[idxd.at[jj - 1]],
                                              ss[(jj - 1) % 2], add=True)
        gd[EBLK - 1].wait()
        sd[EBLK - 1] = pltpu.async_copy(rows[(EBLK - 1) % 2],
                                        acc_sh.at[idxd.at[EBLK - 1]],
                                        ss[(EBLK - 1) % 2], add=True)
        sd[EBLK - 2].wait()
        sd[EBLK - 1].wait()
        return carry

    lax.fori_loop(0, ER_PER_TILE // EBLK, body, 0)
    plsc.subcore_barrier()
    pltpu.sync_copy(acc_sh.at[pl.ds(row0, ROWS_PER_TILE)],
                    out_hbm.at[c, pl.ds(row0, ROWS_PER_TILE)])


def _sc_pairgather_body(u_hbm, v_hbm, qs_hbm, qd_hbm, gu_hbm, gv_hbm,
                        idxs, idxd, ru0, ru1, rv0, rv1,
                        sgu0, sgu1, sgv0, sgv1, swu0, swu1, swv0, swv1):
    wid = _wid()
    base = wid * QR_PER_TILE
    pltpu.sync_copy(qs_hbm.at[pl.ds(base * 128, QR_PER_TILE * 128)], idxs)
    pltpu.sync_copy(qd_hbm.at[pl.ds(base * 128, QR_PER_TILE * 128)], idxd)
    ru = (ru0, ru1)
    rv = (rv0, rv1)
    sgu = (sgu0, sgu1)
    sgv = (sgv0, sgv1)
    swu = (swu0, swu1)
    swv = (swv0, swv1)
    def do_row(j, sl):
        g_u = pltpu.async_copy(u_hbm.at[idxs.at[pl.ds(j * 128, 128)]],
                               ru[sl], sgu[sl])
        g_v = pltpu.async_copy(v_hbm.at[idxd.at[pl.ds(j * 128, 128)]],
                               rv[sl], sgv[sl])
        return g_u, g_v

    def write_row(j, sl):
        w_u = pltpu.async_copy(ru[sl], gu_hbm.at[pl.ds((base + j) * 128, 128)],
                               swu[sl])
        w_v = pltpu.async_copy(rv[sl], gv_hbm.at[pl.ds((base + j) * 128, 128)],
                               swv[sl])
        return w_u, w_v

    def body(i, carry):
        j0 = 2 * i
        j1 = 2 * i + 1
        gu0, gv0 = do_row(j0, 0)
        gu1, gv1 = do_row(j1, 1)
        gu0.wait()
        gv0.wait()
        wu0, wv0 = write_row(j0, 0)
        gu1.wait()
        gv1.wait()
        wu1, wv1 = write_row(j1, 1)
        wu0.wait()
        wv0.wait()
        wu1.wait()
        wv1.wait()
        return carry

    lax.fori_loop(0, QR_PER_TILE // 2, body, 0)
    jt = QR_PER_TILE - 1
    gu_t, gv_t = do_row(jt, 0)
    gu_t.wait()
    gv_t.wait()
    wu_t, wv_t = write_row(jt, 0)
    wu_t.wait()
    wv_t.wait()


@functools.cache
def _sc_kernels():
    mesh = plsc.VectorSubcoreMesh(core_axis_name="c", subcore_axis_name="s",
                                  num_cores=NC, num_subcores=NS)
    sc_degree = pl.kernel(
        _sc_degree_body,
        out_type=jax.ShapeDtypeStruct((NC, N_PAD), jnp.float32),
        mesh=mesh,
        scratch_types=[
            pltpu.VMEM((ER_PER_TILE, 128), jnp.int32),
            pltpu.VMEM((128,), jnp.float32),
            pltpu.VMEM((DEG_PER_TILE,), jnp.float32),
            pltpu.VMEM_SHARED((N_PAD,), jnp.float32),
            pltpu.SemaphoreType.DMA,
        ],
    )
    sc_segsum = pl.kernel(
        _sc_segsum_body,
        out_type=jax.ShapeDtypeStruct((NC, N_PAD, D), jnp.float32),
        mesh=mesh,
        scratch_types=[
            pltpu.VMEM((EBLK, 128), jnp.int32),
            pltpu.VMEM((EBLK, 128), jnp.int32),
            pltpu.VMEM((128, D), jnp.float32),
            pltpu.VMEM((128, D), jnp.float32),
            pltpu.VMEM_SHARED((N_PAD, D), jnp.float32),
            pltpu.SemaphoreType.DMA,
            pltpu.SemaphoreType.DMA,
            pltpu.SemaphoreType.DMA,
            pltpu.SemaphoreType.DMA,
        ],
    )
    sc_pairgather = pl.kernel(
        _sc_pairgather_body,
        out_type=[
            jax.ShapeDtypeStruct((Q_PAD, D), jnp.float32),
            jax.ShapeDtypeStruct((Q_PAD, D), jnp.float32),
        ],
        mesh=mesh,
        scratch_types=(
            [
                pltpu.VMEM((QR_PER_TILE * 128,), jnp.int32),
                pltpu.VMEM((QR_PER_TILE * 128,), jnp.int32),
                pltpu.VMEM((128, D), jnp.float32),
                pltpu.VMEM((128, D), jnp.float32),
                pltpu.VMEM((128, D), jnp.float32),
                pltpu.VMEM((128, D), jnp.float32),
            ]
            + [pltpu.SemaphoreType.DMA] * 8
        ),
    )
    return sc_degree, sc_segsum, sc_pairgather


_BN = 1024
_BQ = 2048


def _tc_prescale_body(x_ref, w1_ref, degp_ref, y1_ref, dinv_ref):
    deg = degp_ref[0, :] + degp_ref[1, :] + 1.0
    dinv = lax.rsqrt(deg)
    xw = jnp.dot(x_ref[...], w1_ref[...], preferred_element_type=jnp.float32)
    y1_ref[...] = xw * dinv[:, None]
    dinv_ref[...] = dinv


def _tc_mid_body(y1_ref, p0_ref, p1_ref, dinv_ref, b1_ref, w2_ref, y2_ref):
    dinv = dinv_ref[...]
    h = (p0_ref[...] + p1_ref[...] + y1_ref[...]) * dinv[:, None] + b1_ref[...][None, :]
    h = jnp.maximum(h, 0.0)
    y2_ref[...] = jnp.dot(h, w2_ref[...], preferred_element_type=jnp.float32) * dinv[:, None]


def _tc_final_body(y2_ref, q0_ref, q1_ref, dinv_ref, b2_ref, wfc_ref, bfc_ref,
                   wd1a_ref, wd1b_ref, bd1_ref, u_ref, v_ref):
    dinv = dinv_ref[...]
    h = (q0_ref[...] + q1_ref[...] + y2_ref[...]) * dinv[:, None] + b2_ref[...][None, :]
    h = jnp.maximum(h, 0.0)
    z = jnp.dot(h, wfc_ref[...], preferred_element_type=jnp.float32) + bfc_ref[...][None, :]
    u_ref[...] = jnp.dot(z, wd1a_ref[...], preferred_element_type=jnp.float32) + bd1_ref[...][None, :]
    v_ref[...] = jnp.dot(z, wd1b_ref[...], preferred_element_type=jnp.float32)


def _tc_dec_body(gu_ref, gv_ref, wd2_ref, bd2_ref, wd3_ref, bd3_ref, o_ref):
    t = jnp.maximum(gu_ref[...] + gv_ref[...], 0.0)
    t2 = jnp.dot(t, wd2_ref[...], preferred_element_type=jnp.float32) + bd2_ref[...][None, :]
    t2 = jnp.maximum(t2, 0.0)
    sc = jnp.sum(t2 * wd3_ref[...][None, :], axis=1) + bd3_ref[...]
    o_ref[...] = jax.nn.sigmoid(sc)


def _full(shape):
    return pl.BlockSpec(shape, lambda i: tuple(0 for _ in shape))


def kernel(x, edge_index, query_edges, W1, b1, W2, b2, Wfc, bfc,
           Wd1, bd1, Wd2, bd2, Wd3, bd3):
    src = edge_index[0].astype(jnp.int32)
    dst = edge_index[1].astype(jnp.int32)
    qs = query_edges[0].astype(jnp.int32)
    qd = query_edges[1].astype(jnp.int32)

    epad = E_PAD - N_EDGES
    pad_src = jnp.arange(epad, dtype=jnp.int32) % N_PAD
    pad_dst = (jnp.arange(epad, dtype=jnp.int32) % (N_PAD - N_NODES)) + N_NODES
    src2d = jnp.concatenate([src, pad_src]).reshape(E_ROWS, 128)
    dst2d = jnp.concatenate([dst, pad_dst]).reshape(E_ROWS, 128)
    qpad = Q_PAD - N_QUERY
    pad_q = jnp.arange(qpad, dtype=jnp.int32) % N_PAD
    qs1d = jnp.concatenate([qs, pad_q])
    qd1d = jnp.concatenate([qd, pad_q])
    ones_row = jnp.ones((128,), jnp.float32)
    xp = jnp.pad(x, ((0, N_PAD - N_NODES), (0, 0)))
    zeros2d = jnp.zeros((N_PAD, D), jnp.float32)

    _sc_degree, _sc_segsum, _sc_pairgather = _sc_kernels()

    degp = _sc_degree(dst2d, ones_row)

    grid_n = N_PAD // _BN
    y1, dinv = pl.pallas_call(
        _tc_prescale_body,
        grid=(grid_n,),
        in_specs=[
            pl.BlockSpec((_BN, D), lambda i: (i, 0)),
            _full((D, D)),
            pl.BlockSpec((NC, _BN), lambda i: (0, i)),
        ],
        out_specs=[
            pl.BlockSpec((_BN, D), lambda i: (i, 0)),
            pl.BlockSpec((_BN,), lambda i: (i,)),
        ],
        out_shape=[
            jax.ShapeDtypeStruct((N_PAD, D), jnp.float32),
            jax.ShapeDtypeStruct((N_PAD,), jnp.float32),
        ],
    )(xp, W1, degp)

    p = _sc_segsum(y1, src2d, dst2d, zeros2d)

    y2 = pl.pallas_call(
        _tc_mid_body,
        grid=(grid_n,),
        in_specs=[
            pl.BlockSpec((_BN, D), lambda i: (i, 0)),
            pl.BlockSpec((_BN, D), lambda i: (i, 0)),
            pl.BlockSpec((_BN, D), lambda i: (i, 0)),
            pl.BlockSpec((_BN,), lambda i: (i,)),
            _full((D,)),
            _full((D, D)),
        ],
        out_specs=pl.BlockSpec((_BN, D), lambda i: (i, 0)),
        out_shape=jax.ShapeDtypeStruct((N_PAD, D), jnp.float32),
    )(y1, p[0], p[1], dinv, b1, W2)

    q = _sc_segsum(y2, src2d, dst2d, zeros2d)

    Wd1a = Wd1[:D]
    Wd1b = Wd1[D:]
    u, v = pl.pallas_call(
        _tc_final_body,
        grid=(grid_n,),
        in_specs=[
            pl.BlockSpec((_BN, D), lambda i: (i, 0)),
            pl.BlockSpec((_BN, D), lambda i: (i, 0)),
            pl.BlockSpec((_BN, D), lambda i: (i, 0)),
            pl.BlockSpec((_BN,), lambda i: (i,)),
            _full((D,)),
            _full((D, D)),
            _full((D,)),
            _full((D, D)),
            _full((D, D)),
            _full((D,)),
        ],
        out_specs=[
            pl.BlockSpec((_BN, D), lambda i: (i, 0)),
            pl.BlockSpec((_BN, D), lambda i: (i, 0)),
        ],
        out_shape=[
            jax.ShapeDtypeStruct((N_PAD, D), jnp.float32),
            jax.ShapeDtypeStruct((N_PAD, D), jnp.float32),
        ],
    )(y2, q[0], q[1], dinv, b2, Wfc, bfc, Wd1a, Wd1b, bd1)

    gu, gv = _sc_pairgather(u, v, qs1d, qd1d)

    grid_q = Q_PAD // _BQ
    out = pl.pallas_call(
        _tc_dec_body,
        grid=(grid_q,),
        in_specs=[
            pl.BlockSpec((_BQ, D), lambda i: (i, 0)),
            pl.BlockSpec((_BQ, D), lambda i: (i, 0)),
            _full((D, D // 2)),
            _full((D // 2,)),
            _full((D // 2,)),
            _full((1,)),
        ],
        out_specs=pl.BlockSpec((_BQ,), lambda i: (i,)),
        out_shape=jax.ShapeDtypeStruct((Q_PAD,), jnp.float32),
    )(gu, gv, Wd2, bd2, Wd3[:, 0], bd3)

    return out[:N_QUERY]


# speedup vs baseline: 2.9933x; 1.0231x over previous
import functools

import jax
import jax.numpy as jnp
from jax import lax
from jax.experimental import pallas as pl
from jax.experimental.pallas import tpu as pltpu
from jax.experimental.pallas import tpu_sc as plsc

N_NODES = 10000
N_PAD = 10240
N_EDGES = 320000
N_QUERY = 100000
D = 128

NC = 2
NS = 16
NW = NC * NS

E_PAD = 327680
E_ROWS = E_PAD // 128
ER_PER_TILE = E_ROWS // NW
EBLK = 8
ROWS_PER_TILE = N_PAD // NS

DEG_PER_TILE = N_PAD // NS

Q_HALF = 53248
Q_PAD = 2 * Q_HALF
QH_ROWS = Q_HALF // 128
QR_PER_TILE = QH_ROWS // NW


def _wid():
    return lax.axis_index("s") * NC + lax.axis_index("c")


def _sc_degree_body(dst2d_hbm, ones_hbm, out_hbm, idxb, onesv, zv, acc_sh, sdeg):
    c = lax.axis_index("c")
    s = lax.axis_index("s")
    wid = _wid()
    for k in range(0, DEG_PER_TILE, 16):
        zv[pl.ds(k, 16)] = jnp.zeros((16,), jnp.float32)
    pltpu.sync_copy(zv, acc_sh.at[pl.ds(s * DEG_PER_TILE, DEG_PER_TILE)])
    pltpu.sync_copy(ones_hbm, onesv)
    plsc.subcore_barrier()

    base = wid * ER_PER_TILE
    pltpu.sync_copy(dst2d_hbm.at[pl.ds(base, ER_PER_TILE)], idxb)
    descs = [
        pltpu.async_copy(onesv, acc_sh.at[idxb.at[j]], sdeg, add=True)
        for j in range(ER_PER_TILE)
    ]
    for d in descs:
        d.wait()
    plsc.subcore_barrier()
    pltpu.sync_copy(
        acc_sh.at[pl.ds(s * DEG_PER_TILE, DEG_PER_TILE)],
        out_hbm.at[c, pl.ds(s * DEG_PER_TILE, DEG_PER_TILE)],
    )


def _sc_segsum_body(y_hbm, src2d_hbm, dst2d_hbm, z_hbm, out_hbm,
                    idxs, idxd, rows0, rows1, acc_sh, sg0, sg1, ss0, ss1):
    c = lax.axis_index("c")
    s = lax.axis_index("s")
    wid = _wid()
    row0 = s * ROWS_PER_TILE
    pltpu.sync_copy(z_hbm.at[pl.ds(row0, ROWS_PER_TILE)],
                    acc_sh.at[pl.ds(row0, ROWS_PER_TILE)])
    plsc.subcore_barrier()

    base = wid * ER_PER_TILE
    rows = (rows0, rows1)
    sg = (sg0, sg1)

    ss = (ss0, ss1)

    def body(i, carry):
        rb = base + i * EBLK
        pltpu.sync_copy(src2d_hbm.at[pl.ds(rb, EBLK)], idxs)
        pltpu.sync_copy(dst2d_hbm.at[pl.ds(rb, EBLK)], idxd)
        gd = [None] * EBLK
        sd = [None] * EBLK
        for jj in range(EBLK):
            sl = jj % 2
            if jj >= 2:
                sd[jj - 2].wait()
            gd[jj] = pltpu.async_copy(y_hbm.at[idxs.at[jj]], rows[sl], sg[sl])
            if jj >= 1:
                gd[jj - 1].wait()
                sd[jj - 1] = pltpu.async_copy(rows[(jj - 1) % 2],
                                              acc_sh.at[idxd.at[jj - 1]],
                                              ss[(jj - 1) % 2], add=True)
        gd[EBLK - 1].wait()
        sd[EBLK - 1] = pltpu.async_copy(rows[(EBLK - 1) % 2],
                                        acc_sh.at[idxd.at[EBLK - 1]],
                                        ss[(EBLK - 1) % 2], add=True)
        sd[EBLK - 2].wait()
        sd[EBLK - 1].wait()
        return carry

    lax.fori_loop(0, ER_PER_TILE // EBLK, body, 0)
    plsc.subcore_barrier()
    pltpu.sync_copy(acc_sh.at[pl.ds(row0, ROWS_PER_TILE)],
                    out_hbm.at[c, pl.ds(row0, ROWS_PER_TILE)])


def _sc_pairgather_body(u_hbm, v_hbm, qs_hbm, qd_hbm, gu_hbm, gv_hbm,
                        idxs, idxd, ru0, ru1, rv0, rv1,
                        sgu0, sgu1, sgv0, sgv1, swu0, swu1, swv0, swv1):
    wid = _wid()
    base = wid * QR_PER_TILE
    pltpu.sync_copy(qs_hbm.at[pl.ds(base * 128, QR_PER_TILE * 128)], idxs)
    pltpu.sync_copy(qd_hbm.at[pl.ds(base * 128, QR_PER_TILE * 128)], idxd)
    ru = (ru0, ru1)
    rv = (rv0, rv1)
    sgu = (sgu0, sgu1)
    sgv = (sgv0, sgv1)
    swu = (swu0, swu1)
    swv = (swv0, swv1)
    def do_row(j, sl):
        g_u = pltpu.async_copy(u_hbm.at[idxs.at[pl.ds(j * 128, 128)]],
                               ru[sl], sgu[sl])
        g_v = pltpu.async_copy(v_hbm.at[idxd.at[pl.ds(j * 128, 128)]],
                               rv[sl], sgv[sl])
        return g_u, g_v

    def write_row(j, sl):
        w_u = pltpu.async_copy(ru[sl], gu_hbm.at[pl.ds((base + j) * 128, 128)],
                               swu[sl])
        w_v = pltpu.async_copy(rv[sl], gv_hbm.at[pl.ds((base + j) * 128, 128)],
                               swv[sl])
        return w_u, w_v

    def body(i, carry):
        j0 = 2 * i
        j1 = 2 * i + 1
        gu0, gv0 = do_row(j0, 0)
        gu1, gv1 = do_row(j1, 1)
        gu0.wait()
        gv0.wait()
        wu0, wv0 = write_row(j0, 0)
        gu1.wait()
        gv1.wait()
        wu1, wv1 = write_row(j1, 1)
        wu0.wait()
        wv0.wait()
        wu1.wait()
        wv1.wait()
        return carry

    lax.fori_loop(0, QR_PER_TILE // 2, body, 0)
    if QR_PER_TILE % 2:
        jt = QR_PER_TILE - 1
        gu_t, gv_t = do_row(jt, 0)
        gu_t.wait()
        gv_t.wait()
        wu_t, wv_t = write_row(jt, 0)
        wu_t.wait()
        wv_t.wait()


@functools.cache
def _sc_kernels():
    mesh = plsc.VectorSubcoreMesh(core_axis_name="c", subcore_axis_name="s",
                                  num_cores=NC, num_subcores=NS)
    sc_degree = pl.kernel(
        _sc_degree_body,
        out_type=jax.ShapeDtypeStruct((NC, N_PAD), jnp.float32),
        mesh=mesh,
        scratch_types=[
            pltpu.VMEM((ER_PER_TILE, 128), jnp.int32),
            pltpu.VMEM((128,), jnp.float32),
            pltpu.VMEM((DEG_PER_TILE,), jnp.float32),
            pltpu.VMEM_SHARED((N_PAD,), jnp.float32),
            pltpu.SemaphoreType.DMA,
        ],
    )
    sc_segsum = pl.kernel(
        _sc_segsum_body,
        out_type=jax.ShapeDtypeStruct((NC, N_PAD, D), jnp.float32),
        mesh=mesh,
        scratch_types=[
            pltpu.VMEM((EBLK, 128), jnp.int32),
            pltpu.VMEM((EBLK, 128), jnp.int32),
            pltpu.VMEM((128, D), jnp.float32),
            pltpu.VMEM((128, D), jnp.float32),
            pltpu.VMEM_SHARED((N_PAD, D), jnp.float32),
            pltpu.SemaphoreType.DMA,
            pltpu.SemaphoreType.DMA,
            pltpu.SemaphoreType.DMA,
            pltpu.SemaphoreType.DMA,
        ],
    )
    sc_pairgather = pl.kernel(
        _sc_pairgather_body,
        out_type=[
            jax.ShapeDtypeStruct((Q_HALF, D), jnp.float32),
            jax.ShapeDtypeStruct((Q_HALF, D), jnp.float32),
        ],
        mesh=mesh,
        scratch_types=(
            [
                pltpu.VMEM((QR_PER_TILE * 128,), jnp.int32),
                pltpu.VMEM((QR_PER_TILE * 128,), jnp.int32),
                pltpu.VMEM((128, D), jnp.float32),
                pltpu.VMEM((128, D), jnp.float32),
                pltpu.VMEM((128, D), jnp.float32),
                pltpu.VMEM((128, D), jnp.float32),
            ]
            + [pltpu.SemaphoreType.DMA] * 8
        ),
    )
    return sc_degree, sc_segsum, sc_pairgather


_BN = 1024
_BQ = 2048


def _tc_prescale_body(x_ref, w1_ref, degp_ref, y1_ref, dinv_ref):
    deg = degp_ref[0, :] + degp_ref[1, :] + 1.0
    dinv = lax.rsqrt(deg)
    xw = jnp.dot(x_ref[...], w1_ref[...], preferred_element_type=jnp.float32)
    y1_ref[...] = xw * dinv[:, None]
    dinv_ref[...] = dinv


def _tc_mid_body(y1_ref, p0_ref, p1_ref, dinv_ref, b1_ref, w2_ref, y2_ref):
    dinv = dinv_ref[...]
    h = (p0_ref[...] + p1_ref[...] + y1_ref[...]) * dinv[:, None] + b1_ref[...][None, :]
    h = jnp.maximum(h, 0.0)
    y2_ref[...] = jnp.dot(h, w2_ref[...], preferred_element_type=jnp.float32) * dinv[:, None]


def _tc_final_body(y2_ref, q0_ref, q1_ref, dinv_ref, b2_ref, wfc_ref, bfc_ref,
                   wd1a_ref, wd1b_ref, bd1_ref, u_ref, v_ref):
    dinv = dinv_ref[...]
    h = (q0_ref[...] + q1_ref[...] + y2_ref[...]) * dinv[:, None] + b2_ref[...][None, :]
    h = jnp.maximum(h, 0.0)
    z = jnp.dot(h, wfc_ref[...], preferred_element_type=jnp.float32) + bfc_ref[...][None, :]
    u_ref[...] = jnp.dot(z, wd1a_ref[...], preferred_element_type=jnp.float32) + bd1_ref[...][None, :]
    v_ref[...] = jnp.dot(z, wd1b_ref[...], preferred_element_type=jnp.float32)


def _tc_dec_body(gu_ref, gv_ref, wd2_ref, bd2_ref, wd3_ref, bd3_ref, o_ref):
    t = jnp.maximum(gu_ref[...] + gv_ref[...], 0.0)
    t2 = jnp.dot(t, wd2_ref[...], preferred_element_type=jnp.float32) + bd2_ref[...][None, :]
    t2 = jnp.maximum(t2, 0.0)
    sc = jnp.sum(t2 * wd3_ref[...][None, :], axis=1) + bd3_ref[...]
    o_ref[...] = jax.nn.sigmoid(sc)


def _full(shape):
    return pl.BlockSpec(shape, lambda i: tuple(0 for _ in shape))


def kernel(x, edge_index, query_edges, W1, b1, W2, b2, Wfc, bfc,
           Wd1, bd1, Wd2, bd2, Wd3, bd3):
    src = edge_index[0].astype(jnp.int32)
    dst = edge_index[1].astype(jnp.int32)
    qs = query_edges[0].astype(jnp.int32)
    qd = query_edges[1].astype(jnp.int32)

    epad = E_PAD - N_EDGES
    pad_src = jnp.arange(epad, dtype=jnp.int32) % N_PAD
    pad_dst = (jnp.arange(epad, dtype=jnp.int32) % (N_PAD - N_NODES)) + N_NODES
    src2d = jnp.concatenate([src, pad_src]).reshape(E_ROWS, 128)
    dst2d = jnp.concatenate([dst, pad_dst]).reshape(E_ROWS, 128)
    qpad = Q_PAD - N_QUERY
    pad_q = jnp.arange(qpad, dtype=jnp.int32) % N_PAD
    qs1d = jnp.concatenate([qs, pad_q])
    qd1d = jnp.concatenate([qd, pad_q])
    ones_row = jnp.ones((128,), jnp.float32)
    xp = jnp.pad(x, ((0, N_PAD - N_NODES), (0, 0)))
    zeros2d = jnp.zeros((N_PAD, D), jnp.float32)

    _sc_degree, _sc_segsum, _sc_pairgather = _sc_kernels()

    degp = _sc_degree(dst2d, ones_row)

    grid_n = N_PAD // _BN
    y1, dinv = pl.pallas_call(
        _tc_prescale_body,
        grid=(grid_n,),
        in_specs=[
            pl.BlockSpec((_BN, D), lambda i: (i, 0)),
            _full((D, D)),
            pl.BlockSpec((NC, _BN), lambda i: (0, i)),
        ],
        out_specs=[
            pl.BlockSpec((_BN, D), lambda i: (i, 0)),
            pl.BlockSpec((_BN,), lambda i: (i,)),
        ],
        out_shape=[
            jax.ShapeDtypeStruct((N_PAD, D), jnp.float32),
            jax.ShapeDtypeStruct((N_PAD,), jnp.float32),
        ],
    )(xp, W1, degp)

    p = _sc_segsum(y1, src2d, dst2d, zeros2d)

    y2 = pl.pallas_call(
        _tc_mid_body,
        grid=(grid_n,),
        in_specs=[
            pl.BlockSpec((_BN, D), lambda i: (i, 0)),
            pl.BlockSpec((_BN, D), lambda i: (i, 0)),
            pl.BlockSpec((_BN, D), lambda i: (i, 0)),
            pl.BlockSpec((_BN,), lambda i: (i,)),
            _full((D,)),
            _full((D, D)),
        ],
        out_specs=pl.BlockSpec((_BN, D), lambda i: (i, 0)),
        out_shape=jax.ShapeDtypeStruct((N_PAD, D), jnp.float32),
    )(y1, p[0], p[1], dinv, b1, W2)

    q = _sc_segsum(y2, src2d, dst2d, zeros2d)

    Wd1a = Wd1[:D]
    Wd1b = Wd1[D:]
    u, v = pl.pallas_call(
        _tc_final_body,
        grid=(grid_n,),
        in_specs=[
            pl.BlockSpec((_BN, D), lambda i: (i, 0)),
            pl.BlockSpec((_BN, D), lambda i: (i, 0)),
            pl.BlockSpec((_BN, D), lambda i: (i, 0)),
            pl.BlockSpec((_BN,), lambda i: (i,)),
            _full((D,)),
            _full((D, D)),
            _full((D,)),
            _full((D, D)),
            _full((D, D)),
            _full((D,)),
        ],
        out_specs=[
            pl.BlockSpec((_BN, D), lambda i: (i, 0)),
            pl.BlockSpec((_BN, D), lambda i: (i, 0)),
        ],
        out_shape=[
            jax.ShapeDtypeStruct((N_PAD, D), jnp.float32),
            jax.ShapeDtypeStruct((N_PAD, D), jnp.float32),
        ],
    )(y2, q[0], q[1], dinv, b2, Wfc, bfc, Wd1a, Wd1b, bd1)

    grid_q = Q_HALF // _BQ
    outs = []
    for h in range(2):
        lo = h * Q_HALF
        gu, gv = _sc_pairgather(u, v, qs1d[lo:lo + Q_HALF],
                                qd1d[lo:lo + Q_HALF])
        outs.append(pl.pallas_call(
            _tc_dec_body,
            grid=(grid_q,),
            in_specs=[
                pl.BlockSpec((_BQ, D), lambda i: (i, 0)),
                pl.BlockSpec((_BQ, D), lambda i: (i, 0)),
                _full((D, D // 2)),
                _full((D // 2,)),
                _full((D // 2,)),
                _full((1,)),
            ],
            out_specs=pl.BlockSpec((_BQ,), lambda i: (i,)),
            out_shape=jax.ShapeDtypeStruct((Q_HALF,), jnp.float32),
        )(gu, gv, Wd2, bd2, Wd3[:, 0], bd3))

    return jnp.concatenate(outs)[:N_QUERY]


# speedup vs baseline: 3.1783x; 1.0618x over previous
import functools

import jax
import jax.numpy as jnp
from jax import lax
from jax.experimental import pallas as pl
from jax.experimental.pallas import tpu as pltpu
from jax.experimental.pallas import tpu_sc as plsc

N_NODES = 10000
N_PAD = 10240
N_EDGES = 320000
N_QUERY = 100000
D = 128

NC = 2
NS = 16
NW = NC * NS

E_PAD = 327680
E_ROWS = E_PAD // 128
ER_PER_TILE = E_ROWS // NW
EBLK = 16
ROWS_PER_TILE = N_PAD // NS

DEG_PER_TILE = N_PAD // NS

Q_HALF = 53248
Q_PAD = 2 * Q_HALF
QH_ROWS = Q_HALF // 128
QR_PER_TILE = QH_ROWS // NW


def _wid():
    return lax.axis_index("s") * NC + lax.axis_index("c")


def _sc_degree_body(dst2d_hbm, ones_hbm, out_hbm, idxb, onesv, zv, acc_sh, sdeg):
    c = lax.axis_index("c")
    s = lax.axis_index("s")
    wid = _wid()
    for k in range(0, DEG_PER_TILE, 16):
        zv[pl.ds(k, 16)] = jnp.zeros((16,), jnp.float32)
    pltpu.sync_copy(zv, acc_sh.at[pl.ds(s * DEG_PER_TILE, DEG_PER_TILE)])
    pltpu.sync_copy(ones_hbm, onesv)
    plsc.subcore_barrier()

    base = wid * ER_PER_TILE
    pltpu.sync_copy(dst2d_hbm.at[pl.ds(base, ER_PER_TILE)], idxb)
    descs = [
        pltpu.async_copy(onesv, acc_sh.at[idxb.at[j]], sdeg, add=True)
        for j in range(ER_PER_TILE)
    ]
    for d in descs:
        d.wait()
    plsc.subcore_barrier()
    pltpu.sync_copy(
        acc_sh.at[pl.ds(s * DEG_PER_TILE, DEG_PER_TILE)],
        out_hbm.at[c, pl.ds(s * DEG_PER_TILE, DEG_PER_TILE)],
    )


def _sc_segsum_body(y_hbm, src2d_hbm, dst2d_hbm, z_hbm, out_hbm,
                    idxs, idxd, rows0, rows1, acc_sh, sg0, sg1, ss0, ss1):
    c = lax.axis_index("c")
    s = lax.axis_index("s")
    wid = _wid()
    row0 = s * ROWS_PER_TILE
    pltpu.sync_copy(z_hbm.at[pl.ds(row0, ROWS_PER_TILE)],
                    acc_sh.at[pl.ds(row0, ROWS_PER_TILE)])
    plsc.subcore_barrier()

    base = wid * ER_PER_TILE
    rows = (rows0, rows1)
    sg = (sg0, sg1)

    ss = (ss0, ss1)

    def body(i, carry):
        rb = base + i * EBLK
        pltpu.sync_copy(src2d_hbm.at[pl.ds(rb, EBLK)], idxs)
        pltpu.sync_copy(dst2d_hbm.at[pl.ds(rb, EBLK)], idxd)
        gd = [None] * EBLK
        sd = [None] * EBLK
        for jj in range(EBLK):
            sl = jj % 2
            if jj >= 2:
                sd[jj - 2].wait()
            gd[jj] = pltpu.async_copy(y_hbm.at[idxs.at[jj]], rows[sl], sg[sl])
            if jj >= 1:
                gd[jj - 1].wait()
                sd[jj - 1] = pltpu.async_copy(rows[(jj - 1) % 2],
                                              acc_sh.at[idxd.at[jj - 1]],
                                              ss[(jj - 1) % 2], add=True)
        gd[EBLK - 1].wait()
        sd[EBLK - 1] = pltpu.async_copy(rows[(EBLK - 1) % 2],
                                        acc_sh.at[idxd.at[EBLK - 1]],
                                        ss[(EBLK - 1) % 2], add=True)
        sd[EBLK - 2].wait()
        sd[EBLK - 1].wait()
        return carry

    lax.fori_loop(0, ER_PER_TILE // EBLK, body, 0)
    plsc.subcore_barrier()
    pltpu.sync_copy(acc_sh.at[pl.ds(row0, ROWS_PER_TILE)],
                    out_hbm.at[c, pl.ds(row0, ROWS_PER_TILE)])


def _sc_pairgather_body(u_hbm, v_hbm, qs_hbm, qd_hbm, gu_hbm, gv_hbm,
                        idxs, idxd, ru0, ru1, rv0, rv1,
                        sgu0, sgu1, sgv0, sgv1, swu0, swu1, swv0, swv1):
    wid = _wid()
    base = wid * QR_PER_TILE
    pltpu.sync_copy(qs_hbm.at[pl.ds(base * 128, QR_PER_TILE * 128)], idxs)
    pltpu.sync_copy(qd_hbm.at[pl.ds(base * 128, QR_PER_TILE * 128)], idxd)
    ru = (ru0, ru1)
    rv = (rv0, rv1)
    sgu = (sgu0, sgu1)
    sgv = (sgv0, sgv1)
    swu = (swu0, swu1)
    swv = (swv0, swv1)
    def do_row(j, sl):
        g_u = pltpu.async_copy(u_hbm.at[idxs.at[pl.ds(j * 128, 128)]],
                               ru[sl], sgu[sl])
        g_v = pltpu.async_copy(v_hbm.at[idxd.at[pl.ds(j * 128, 128)]],
                               rv[sl], sgv[sl])
        return g_u, g_v

    def write_row(j, sl):
        w_u = pltpu.async_copy(ru[sl], gu_hbm.at[pl.ds((base + j) * 128, 128)],
                               swu[sl])
        w_v = pltpu.async_copy(rv[sl], gv_hbm.at[pl.ds((base + j) * 128, 128)],
                               swv[sl])
        return w_u, w_v

    def body(i, carry):
        j0 = 2 * i
        j1 = 2 * i + 1
        gu0, gv0 = do_row(j0, 0)
        gu1, gv1 = do_row(j1, 1)
        gu0.wait()
        gv0.wait()
        wu0, wv0 = write_row(j0, 0)
        gu1.wait()
        gv1.wait()
        wu1, wv1 = write_row(j1, 1)
        wu0.wait()
        wv0.wait()
        wu1.wait()
        wv1.wait()
        return carry

    lax.fori_loop(0, QR_PER_TILE // 2, body, 0)
    if QR_PER_TILE % 2:
        jt = QR_PER_TILE - 1
        gu_t, gv_t = do_row(jt, 0)
        gu_t.wait()
        gv_t.wait()
        wu_t, wv_t = write_row(jt, 0)
        wu_t.wait()
        wv_t.wait()


@functools.cache
def _sc_kernels():
    mesh = plsc.VectorSubcoreMesh(core_axis_name="c", subcore_axis_name="s",
                                  num_cores=NC, num_subcores=NS)
    sc_degree = pl.kernel(
        _sc_degree_body,
        out_type=jax.ShapeDtypeStruct((NC, N_PAD), jnp.float32),
        mesh=mesh,
        scratch_types=[
            pltpu.VMEM((ER_PER_TILE, 128), jnp.int32),
            pltpu.VMEM((128,), jnp.float32),
            pltpu.VMEM((DEG_PER_TILE,), jnp.float32),
            pltpu.VMEM_SHARED((N_PAD,), jnp.float32),
            pltpu.SemaphoreType.DMA,
        ],
    )
    sc_segsum = pl.kernel(
        _sc_segsum_body,
        out_type=jax.ShapeDtypeStruct((NC, N_PAD, D), jnp.float32),
        mesh=mesh,
        scratch_types=[
            pltpu.VMEM((EBLK, 128), jnp.int32),
            pltpu.VMEM((EBLK, 128), jnp.int32),
            pltpu.VMEM((128, D), jnp.float32),
            pltpu.VMEM((128, D), jnp.float32),
            pltpu.VMEM_SHARED((N_PAD, D), jnp.float32),
            pltpu.SemaphoreType.DMA,
            pltpu.SemaphoreType.DMA,
            pltpu.SemaphoreType.DMA,
            pltpu.SemaphoreType.DMA,
        ],
    )
    sc_pairgather = pl.kernel(
        _sc_pairgather_body,
        out_type=[
            jax.ShapeDtypeStruct((Q_HALF, D), jnp.float32),
            jax.ShapeDtypeStruct((Q_HALF, D), jnp.float32),
        ],
        mesh=mesh,
        scratch_types=(
            [
                pltpu.VMEM((QR_PER_TILE * 128,), jnp.int32),
                pltpu.VMEM((QR_PER_TILE * 128,), jnp.int32),
                pltpu.VMEM((128, D), jnp.float32),
                pltpu.VMEM((128, D), jnp.float32),
                pltpu.VMEM((128, D), jnp.float32),
                pltpu.VMEM((128, D), jnp.float32),
            ]
            + [pltpu.SemaphoreType.DMA] * 8
        ),
    )
    return sc_degree, sc_segsum, sc_pairgather


_BN = 1024
_BQ = 4096


def _tc_prescale_body(x_ref, w1_ref, degp_ref, y1_ref, dinv_ref):
    deg = degp_ref[0, :] + degp_ref[1, :] + 1.0
    dinv = lax.rsqrt(deg)
    xw = jnp.dot(x_ref[...], w1_ref[...], preferred_element_type=jnp.float32)
    y1_ref[...] = xw * dinv[:, None]
    dinv_ref[...] = dinv


def _tc_mid_body(y1_ref, p0_ref, p1_ref, dinv_ref, b1_ref, w2_ref, y2_ref):
    dinv = dinv_ref[...]
    h = (p0_ref[...] + p1_ref[...] + y1_ref[...]) * dinv[:, None] + b1_ref[...][None, :]
    h = jnp.maximum(h, 0.0)
    y2_ref[...] = jnp.dot(h, w2_ref[...], preferred_element_type=jnp.float32) * dinv[:, None]


def _tc_final_body(y2_ref, q0_ref, q1_ref, dinv_ref, b2_ref, wfc_ref, bfc_ref,
                   wd1a_ref, wd1b_ref, bd1_ref, u_ref, v_ref):
    dinv = dinv_ref[...]
    h = (q0_ref[...] + q1_ref[...] + y2_ref[...]) * dinv[:, None] + b2_ref[...][None, :]
    h = jnp.maximum(h, 0.0)
    z = jnp.dot(h, wfc_ref[...], preferred_element_type=jnp.float32) + bfc_ref[...][None, :]
    u_ref[...] = jnp.dot(z, wd1a_ref[...], preferred_element_type=jnp.float32) + bd1_ref[...][None, :]
    v_ref[...] = jnp.dot(z, wd1b_ref[...], preferred_element_type=jnp.float32)


def _tc_dec_body(gu_ref, gv_ref, wd2_ref, bd2_ref, wd3_ref, bd3_ref, o_ref):
    t = jnp.maximum(gu_ref[...] + gv_ref[...], 0.0)
    t2 = jnp.dot(t, wd2_ref[...], preferred_element_type=jnp.float32) + bd2_ref[...][None, :]
    t2 = jnp.maximum(t2, 0.0)
    sc = jnp.sum(t2 * wd3_ref[...][None, :], axis=1) + bd3_ref[...]
    o_ref[...] = jax.nn.sigmoid(sc)


def _full(shape):
    return pl.BlockSpec(shape, lambda i: tuple(0 for _ in shape))


def kernel(x, edge_index, query_edges, W1, b1, W2, b2, Wfc, bfc,
           Wd1, bd1, Wd2, bd2, Wd3, bd3):
    src = edge_index[0].astype(jnp.int32)
    dst = edge_index[1].astype(jnp.int32)
    qs = query_edges[0].astype(jnp.int32)
    qd = query_edges[1].astype(jnp.int32)

    epad = E_PAD - N_EDGES
    pad_src = jnp.arange(epad, dtype=jnp.int32) % N_PAD
    pad_dst = (jnp.arange(epad, dtype=jnp.int32) % (N_PAD - N_NODES)) + N_NODES
    src2d = jnp.concatenate([src, pad_src]).reshape(E_ROWS, 128)
    dst2d = jnp.concatenate([dst, pad_dst]).reshape(E_ROWS, 128)
    qpad = Q_PAD - N_QUERY
    pad_q = jnp.arange(qpad, dtype=jnp.int32) % N_PAD
    qs1d = jnp.concatenate([qs, pad_q])
    qd1d = jnp.concatenate([qd, pad_q])
    ones_row = jnp.ones((128,), jnp.float32)
    xp = jnp.pad(x, ((0, N_PAD - N_NODES), (0, 0)))
    zeros2d = jnp.zeros((N_PAD, D), jnp.float32)

    _sc_degree, _sc_segsum, _sc_pairgather = _sc_kernels()

    degp = _sc_degree(dst2d, ones_row)

    grid_n = N_PAD // _BN
    y1, dinv = pl.pallas_call(
        _tc_prescale_body,
        grid=(grid_n,),
        in_specs=[
            pl.BlockSpec((_BN, D), lambda i: (i, 0)),
            _full((D, D)),
            pl.BlockSpec((NC, _BN), lambda i: (0, i)),
        ],
        out_specs=[
            pl.BlockSpec((_BN, D), lambda i: (i, 0)),
            pl.BlockSpec((_BN,), lambda i: (i,)),
        ],
        out_shape=[
            jax.ShapeDtypeStruct((N_PAD, D), jnp.float32),
            jax.ShapeDtypeStruct((N_PAD,), jnp.float32),
        ],
    )(xp, W1, degp)

    p = _sc_segsum(y1, src2d, dst2d, zeros2d)

    y2 = pl.pallas_call(
        _tc_mid_body,
        grid=(grid_n,),
        in_specs=[
            pl.BlockSpec((_BN, D), lambda i: (i, 0)),
            pl.BlockSpec((_BN, D), lambda i: (i, 0)),
            pl.BlockSpec((_BN, D), lambda i: (i, 0)),
            pl.BlockSpec((_BN,), lambda i: (i,)),
            _full((D,)),
            _full((D, D)),
        ],
        out_specs=pl.BlockSpec((_BN, D), lambda i: (i, 0)),
        out_shape=jax.ShapeDtypeStruct((N_PAD, D), jnp.float32),
    )(y1, p[0], p[1], dinv, b1, W2)

    q = _sc_segsum(y2, src2d, dst2d, zeros2d)

    Wd1a = Wd1[:D]
    Wd1b = Wd1[D:]
    u, v = pl.pallas_call(
        _tc_final_body,
        grid=(grid_n,),
        in_specs=[
            pl.BlockSpec((_BN, D), lambda i: (i, 0)),
            pl.BlockSpec((_BN, D), lambda i: (i, 0)),
            pl.BlockSpec((_BN, D), lambda i: (i, 0)),
            pl.BlockSpec((_BN,), lambda i: (i,)),
            _full((D,)),
            _full((D, D)),
            _full((D,)),
            _full((D, D)),
            _full((D, D)),
            _full((D,)),
        ],
        out_specs=[
            pl.BlockSpec((_BN, D), lambda i: (i, 0)),
            pl.BlockSpec((_BN, D), lambda i: (i, 0)),
        ],
        out_shape=[
            jax.ShapeDtypeStruct((N_PAD, D), jnp.float32),
            jax.ShapeDtypeStruct((N_PAD, D), jnp.float32),
        ],
    )(y2, q[0], q[1], dinv, b2, Wfc, bfc, Wd1a, Wd1b, bd1)

    grid_q = Q_HALF // _BQ
    outs = []
    for h in range(2):
        lo = h * Q_HALF
        gu, gv = _sc_pairgather(u, v, qs1d[lo:lo + Q_HALF],
                                qd1d[lo:lo + Q_HALF])
        outs.append(pl.pallas_call(
            _tc_dec_body,
            grid=(grid_q,),
            in_specs=[
                pl.BlockSpec((_BQ, D), lambda i: (i, 0)),
                pl.BlockSpec((_BQ, D), lambda i: (i, 0)),
                _full((D, D // 2)),
                _full((D // 2,)),
                _full((D // 2,)),
                _full((1,)),
            ],
            out_specs=pl.BlockSpec((_BQ,), lambda i: (i,)),
            out_shape=jax.ShapeDtypeStruct((Q_HALF,), jnp.float32),
        )(gu, gv, Wd2, bd2, Wd3[:, 0], bd3))

    return jnp.concatenate(outs)[:N_QUERY]
